# baseline (speedup 1.0000x reference)
import functools

import jax
import jax.numpy as jnp
import numpy as np
from jax import lax
from jax.experimental import pallas as pl
from jax.experimental.pallas import tpu as pltpu
from jax.experimental.pallas import tpu_sc as plsc

N = 10000
E = 160000
D = 256
NG = 2
GW = D // NG
NPAD = 10240
STRIPE = NPAD // 16
NC = 2
NS = 16

AGG_CHUNK = 128
AGG_NCHUNK = 80
AGG_K = 8
AGG_NGRP = AGG_NCHUNK // AGG_K
EP_TILE = AGG_CHUNK * AGG_NCHUNK
E_PAD = EP_TILE * NS

DEG_CHUNK = 40
DEG_NCHUNK = (E // (NC * NS)) // DEG_CHUNK

ROWS = 1000
GRID = N // ROWS

_SQRT_HALF = np.float32(1.0 / np.sqrt(2.0))


def _sc_mesh():
  return plsc.VectorSubcoreMesh(
      core_axis_name="c", subcore_axis_name="s", num_cores=NC,
      num_subcores=NS)


@functools.partial(
    pl.kernel,
    out_type=jax.ShapeDtypeStruct((NC, NPAD), jnp.float32),
    mesh=_sc_mesh(),
    scratch_types=[
        pltpu.VMEM((DEG_NCHUNK, DEG_CHUNK), jnp.int32),
        pltpu.VMEM((DEG_CHUNK,), jnp.float32),
        pltpu.VMEM_SHARED((NPAD,), jnp.float32),
        pltpu.SemaphoreType.DMA,
    ],
)
def _deg_kernel(dst_hbm, ones_hbm, zeros_hbm, out_hbm, idx_v, ones_v,
                cnt_sh, sem):
  c = lax.axis_index("c")
  s = lax.axis_index("s")
  pltpu.sync_copy(dst_hbm.at[c, s], idx_v)
  pltpu.sync_copy(ones_hbm, ones_v)
  pltpu.sync_copy(zeros_hbm, cnt_sh.at[pl.ds(s * STRIPE, STRIPE)])
  plsc.subcore_barrier()

  def step(g, carry):
    base = g * 5
    for k in range(5):
      pltpu.async_copy(ones_v, cnt_sh.at[idx_v.at[base + k]], sem, add=True)
    for k in range(5):
      pltpu.make_async_copy(
          ones_v, cnt_sh.at[idx_v.at[base + k]], sem).wait()
    return carry

  lax.fori_loop(0, DEG_NCHUNK // 5, step, 0)
  plsc.subcore_barrier()
  pltpu.sync_copy(cnt_sh.at[pl.ds(s * STRIPE, STRIPE)],
                  out_hbm.at[c].at[pl.ds(s * STRIPE, STRIPE)])


@functools.partial(
    pl.kernel,
    out_type=jax.ShapeDtypeStruct((NG, NPAD, GW), jnp.float32),
    mesh=_sc_mesh(),
    scratch_types=[
        pltpu.VMEM((2, AGG_CHUNK), jnp.int32),
        pltpu.VMEM((AGG_NCHUNK, AGG_CHUNK), jnp.int32),
        pltpu.VMEM((2, AGG_CHUNK, GW), jnp.float32),
        pltpu.VMEM_SHARED((NPAD, GW), jnp.float32),
        pltpu.SemaphoreType.DMA,
        pltpu.SemaphoreType.DMA,
        pltpu.SemaphoreType.DMA,
    ],
)
def _agg_kernel(hl_hbm, hr_hbm, src_hbm, dstidx_hbm,
                zeros_hbm, out_hbm, sidx_v, didx_v, buf_v, agg_sh,
                gsem, isem, ssem):
  c = lax.axis_index("c")
  s = lax.axis_index("s")
  pltpu.sync_copy(dstidx_hbm.at[s], didx_v)
  pltpu.sync_copy(zeros_hbm, agg_sh.at[pl.ds(s * STRIPE, STRIPE)])
  plsc.subcore_barrier()

  def pipeline(tbl):
    pltpu.sync_copy(src_hbm.at[s, 0], sidx_v.at[0])
    pltpu.async_copy(src_hbm.at[s, 1], sidx_v.at[1], isem)
    pltpu.async_copy(tbl.at[sidx_v.at[0]], buf_v.at[0], gsem)

    def group(g, carry):
      j0 = g * AGG_K
      sdescs = []
      for k in range(AGG_K):
        j = j0 + k
        cur = k % 2
        nxt = 1 - cur
        pltpu.make_async_copy(
            tbl.at[sidx_v.at[cur]], buf_v.at[cur], gsem).wait()
        if k >= 1:
          sdescs[k - 1].wait()

        def start_next():
          pltpu.make_async_copy(
              src_hbm.at[s, j + 1], sidx_v.at[nxt], isem).wait()
          pltpu.async_copy(tbl.at[sidx_v.at[nxt]], buf_v.at[nxt], gsem)

        if k < AGG_K - 1:
          start_next()
        else:
          pl.when(g < AGG_NGRP - 1)(start_next)

        @pl.when(j < AGG_NCHUNK - 2)
        def _():
          pltpu.async_copy(src_hbm.at[s, j + 2], sidx_v.at[cur], isem)

        sdescs.append(
            pltpu.async_copy(buf_v.at[cur], agg_sh.at[didx_v.at[j]],
                             ssem, add=True))
      sdescs[-1].wait()
      return carry

    lax.fori_loop(0, AGG_NGRP, group, 0)

  @pl.when(c == 0)
  def _():
    pipeline(hl_hbm)

  @pl.when(c == 1)
  def _():
    pipeline(hr_hbm)

  plsc.subcore_barrier()
  pltpu.sync_copy(agg_sh.at[pl.ds(s * STRIPE, STRIPE)],
                  out_hbm.at[c].at[pl.ds(s * STRIPE, STRIPE)])


def _dinv_from(degt):
  return lax.rsqrt(jnp.sum(degt, axis=-1, keepdims=True) + 1.0)


def _ln(v, g, b):
  mu = jnp.mean(v, axis=-1, keepdims=True)
  xc = v - mu
  var = jnp.mean(xc * xc, axis=-1, keepdims=True)
  return xc * lax.rsqrt(var + 1e-5) * g + b


def _gelu(v):
  return 0.5 * v * (1.0 + lax.erf(v * _SQRT_HALF))


def _t1_body(x_ref, w_ref, degt_ref, *h_refs):
  h = jnp.dot(x_ref[...], w_ref[...],
              preferred_element_type=jnp.float32,
              precision=lax.Precision.HIGHEST)
  h = h * _dinv_from(degt_ref[...])
  for g in range(NG):
    h_refs[g][...] = h[:, g * GW:(g + 1) * GW]


def _t2_body(a0, a1, h0, h1, degt_ref, b1_ref, g1_ref,
             bb1_ref, w2_ref, *o_refs):
  dinv = _dinv_from(degt_ref[...])
  agg = jnp.concatenate([a0[0], a1[0]], axis=-1)
  hp = jnp.concatenate([h0[...], h1[...]], axis=-1)
  conv = dinv * (agg + hp) + b1_ref[...]
  o1 = _gelu(_ln(conv, g1_ref[...], bb1_ref[...]))
  hh = jnp.dot(o1, w2_ref[...],
               preferred_element_type=jnp.float32,
               precision=lax.Precision.HIGHEST)
  hh = hh * dinv
  for g in range(NG):
    o_refs[g][...] = hh[:, g * GW:(g + 1) * GW]


def _t3_body(a0, a1, h0, h1, degt_ref, b2_ref, g2_ref,
             bb2_ref, x_ref, out_ref):
  dinv = _dinv_from(degt_ref[...])
  agg = jnp.concatenate([a0[0], a1[0]], axis=-1)
  hp = jnp.concatenate([h0[...], h1[...]], axis=-1)
  conv = dinv * (agg + hp) + b2_ref[...]
  out_ref[...] = _gelu(_ln(conv, g2_ref[...], bb2_ref[...]) + x_ref[...])


def _row_spec(cols):
  return pl.BlockSpec((ROWS, cols), lambda i: (i, 0))


def _full_spec(shape):
  nd = len(shape)
  return pl.BlockSpec(shape, lambda i: (0,) * nd)


def _agg_spec(group):
  return pl.BlockSpec((1, ROWS, GW), lambda i, g=group: (g, i, 0))


_DEGT_SPEC = pl.BlockSpec((ROWS, NC), lambda i: (i, 0))
_AGG_SPECS = [_agg_spec(g) for g in range(NG)]
_H_SPECS = [_row_spec(GW)] * NG
_H_SHAPES = [jax.ShapeDtypeStruct((N, GW), jnp.float32)] * NG

_t1 = pl.pallas_call(
    _t1_body,
    grid=(GRID,),
    in_specs=[_row_spec(D), _full_spec((D, D)), _DEGT_SPEC],
    out_specs=_H_SPECS,
    out_shape=_H_SHAPES,
)

_t2 = pl.pallas_call(
    _t2_body,
    grid=(GRID,),
    in_specs=_AGG_SPECS + _H_SPECS +
             [_DEGT_SPEC, _full_spec((D,)), _full_spec((D,)),
              _full_spec((D,)), _full_spec((D, D))],
    out_specs=_H_SPECS,
    out_shape=_H_SHAPES,
)

_t3 = pl.pallas_call(
    _t3_body,
    grid=(GRID,),
    in_specs=_AGG_SPECS + _H_SPECS +
             [_DEGT_SPEC, _full_spec((D,)), _full_spec((D,)),
              _full_spec((D,)), _row_spec(D)],
    out_specs=pl.BlockSpec((ROWS, D), lambda i: (i, 0)),
    out_shape=jax.ShapeDtypeStruct((N, D), jnp.float32),
)


@jax.jit
def kernel(x, edge_index, W1, b1, W2, b2, ln1_g, ln1_b, ln2_g, ln2_b):
  npad = E_PAD - E
  pad_src = jnp.arange(npad, dtype=jnp.int32) % N
  pad_dst = jnp.arange(npad, dtype=jnp.int32) % (NPAD - N) + N
  src = jnp.concatenate([edge_index[0], pad_src]).reshape(
      NS, AGG_NCHUNK, AGG_CHUNK)
  dst = jnp.concatenate([edge_index[1], pad_dst]).reshape(
      NS, AGG_NCHUNK, AGG_CHUNK)
  dst_deg = edge_index[1].reshape(NC, NS, DEG_NCHUNK, DEG_CHUNK)

  ones_c = jnp.ones((DEG_CHUNK,), jnp.float32)
  zeros_1 = jnp.zeros((STRIPE,), jnp.float32)
  zeros_2 = jnp.zeros((STRIPE, GW), jnp.float32)

  degp = _deg_kernel(dst_deg, ones_c, zeros_1)
  degt = degp.T

  h1 = _t1(x, W1, degt)
  agg1 = _agg_kernel(*h1, src, dst, zeros_2)
  h2 = _t2(agg1, agg1, *h1, degt, b1, ln1_g, ln1_b, W2)
  agg2 = _agg_kernel(*h2, src, dst, zeros_2)
  out = _t3(agg2, agg2, *h2, degt, b2, ln2_g, ln2_b, x)
  return out


# speedup vs baseline: 1.0028x; 1.0028x over previous
import functools

import jax
import jax.numpy as jnp
import numpy as np
from jax import lax
from jax.experimental import pallas as pl
from jax.experimental.pallas import tpu as pltpu
from jax.experimental.pallas import tpu_sc as plsc

N = 10000
E = 160000
D = 256
NG = 2
GW = D // NG
NPAD = 10240
STRIPE = NPAD // 16
NC = 2
NS = 16

AGG_CHUNK = 128
AGG_NCHUNK = 80
AGG_K = 8
AGG_NGRP = AGG_NCHUNK // AGG_K
EP_TILE = AGG_CHUNK * AGG_NCHUNK
E_PAD = EP_TILE * NS

DEG_CHUNK = 40
DEG_NCHUNK = (E // (NC * NS)) // DEG_CHUNK

ROWS = 1000
GRID = N // ROWS

_SQRT_HALF = np.float32(1.0 / np.sqrt(2.0))


def _sc_mesh():
  return plsc.VectorSubcoreMesh(
      core_axis_name="c", subcore_axis_name="s", num_cores=NC,
      num_subcores=NS)


@functools.partial(
    pl.kernel,
    out_type=jax.ShapeDtypeStruct((NC, NPAD), jnp.float32),
    mesh=_sc_mesh(),
    scratch_types=[
        pltpu.VMEM((DEG_NCHUNK, DEG_CHUNK), jnp.int32),
        pltpu.VMEM((DEG_CHUNK,), jnp.float32),
        pltpu.VMEM_SHARED((NPAD,), jnp.float32),
        pltpu.SemaphoreType.DMA,
    ],
)
def _deg_kernel(dst_hbm, ones_hbm, zeros_hbm, out_hbm, idx_v, ones_v,
                cnt_sh, sem):
  c = lax.axis_index("c")
  s = lax.axis_index("s")
  pltpu.sync_copy(dst_hbm.at[c, s], idx_v)
  pltpu.sync_copy(ones_hbm, ones_v)
  pltpu.sync_copy(zeros_hbm, cnt_sh.at[pl.ds(s * STRIPE, STRIPE)])
  plsc.subcore_barrier()

  def step(g, carry):
    base = g * 5
    for k in range(5):
      pltpu.async_copy(ones_v, cnt_sh.at[idx_v.at[base + k]], sem, add=True)
    for k in range(5):
      pltpu.make_async_copy(
          ones_v, cnt_sh.at[idx_v.at[base + k]], sem).wait()
    return carry

  lax.fori_loop(0, DEG_NCHUNK // 5, step, 0)
  plsc.subcore_barrier()
  pltpu.sync_copy(cnt_sh.at[pl.ds(s * STRIPE, STRIPE)],
                  out_hbm.at[c].at[pl.ds(s * STRIPE, STRIPE)])


@functools.partial(
    pl.kernel,
    out_type=jax.ShapeDtypeStruct((NG, NPAD, GW), jnp.float32),
    mesh=_sc_mesh(),
    scratch_types=[
        pltpu.VMEM((2, AGG_CHUNK), jnp.int32),
        pltpu.VMEM((AGG_NCHUNK, AGG_CHUNK), jnp.int32),
        pltpu.VMEM((2, AGG_CHUNK, GW), jnp.float32),
        pltpu.VMEM_SHARED((NPAD, GW), jnp.float32),
        pltpu.SemaphoreType.DMA,
        pltpu.SemaphoreType.DMA,
    ],
)
def _agg_kernel(hl_hbm, hr_hbm, src_hbm, dstidx_hbm,
                zeros_hbm, out_hbm, sidx_v, didx_v, buf_v, agg_sh,
                gsem, isem):
  c = lax.axis_index("c")
  s = lax.axis_index("s")
  pltpu.sync_copy(dstidx_hbm.at[s], didx_v)
  pltpu.sync_copy(zeros_hbm, agg_sh.at[pl.ds(s * STRIPE, STRIPE)])
  plsc.subcore_barrier()

  def pipeline(tbl):
    pltpu.sync_copy(src_hbm.at[s, 0], sidx_v.at[0])
    pltpu.async_copy(src_hbm.at[s, 1], sidx_v.at[1], isem)
    pltpu.async_copy(tbl.at[sidx_v.at[0]], buf_v.at[0], gsem)

    def step(j, carry):
      cur = lax.rem(j, 2)
      nxt = 1 - cur
      pltpu.make_async_copy(
          tbl.at[sidx_v.at[cur]], buf_v.at[cur], gsem).wait()

      @pl.when(j < AGG_NCHUNK - 1)
      def _():
        pltpu.make_async_copy(
            src_hbm.at[s, j + 1], sidx_v.at[nxt], isem).wait()
        pltpu.async_copy(tbl.at[sidx_v.at[nxt]], buf_v.at[nxt], gsem)

      @pl.when(j < AGG_NCHUNK - 2)
      def _():
        pltpu.async_copy(src_hbm.at[s, j + 2], sidx_v.at[cur], isem)

      pltpu.sync_copy(buf_v.at[cur], agg_sh.at[didx_v.at[j]], add=True)
      return carry

    lax.fori_loop(0, AGG_NCHUNK, step, 0)

  @pl.when(c == 0)
  def _():
    pipeline(hl_hbm)

  @pl.when(c == 1)
  def _():
    pipeline(hr_hbm)

  plsc.subcore_barrier()
  pltpu.sync_copy(agg_sh.at[pl.ds(s * STRIPE, STRIPE)],
                  out_hbm.at[c].at[pl.ds(s * STRIPE, STRIPE)])


def _dinv_from(degt):
  return lax.rsqrt(jnp.sum(degt, axis=-1, keepdims=True) + 1.0)


def _ln(v, g, b):
  mu = jnp.mean(v, axis=-1, keepdims=True)
  xc = v - mu
  var = jnp.mean(xc * xc, axis=-1, keepdims=True)
  return xc * lax.rsqrt(var + 1e-5) * g + b


def _gelu(v):
  return 0.5 * v * (1.0 + lax.erf(v * _SQRT_HALF))


def _t1_body(x_ref, w_ref, degt_ref, *h_refs):
  h = jnp.dot(x_ref[...], w_ref[...],
              preferred_element_type=jnp.float32,
              precision=lax.Precision.HIGHEST)
  h = h * _dinv_from(degt_ref[...])
  for g in range(NG):
    h_refs[g][...] = h[:, g * GW:(g + 1) * GW]


def _t2_body(a0, a1, h0, h1, degt_ref, b1_ref, g1_ref,
             bb1_ref, w2_ref, *o_refs):
  dinv = _dinv_from(degt_ref[...])
  agg = jnp.concatenate([a0[0], a1[0]], axis=-1)
  hp = jnp.concatenate([h0[...], h1[...]], axis=-1)
  conv = dinv * (agg + hp) + b1_ref[...]
  o1 = _gelu(_ln(conv, g1_ref[...], bb1_ref[...]))
  hh = jnp.dot(o1, w2_ref[...],
               preferred_element_type=jnp.float32,
               precision=lax.Precision.HIGHEST)
  hh = hh * dinv
  for g in range(NG):
    o_refs[g][...] = hh[:, g * GW:(g + 1) * GW]


def _t3_body(a0, a1, h0, h1, degt_ref, b2_ref, g2_ref,
             bb2_ref, x_ref, out_ref):
  dinv = _dinv_from(degt_ref[...])
  agg = jnp.concatenate([a0[0], a1[0]], axis=-1)
  hp = jnp.concatenate([h0[...], h1[...]], axis=-1)
  conv = dinv * (agg + hp) + b2_ref[...]
  out_ref[...] = _gelu(_ln(conv, g2_ref[...], bb2_ref[...]) + x_ref[...])


def _row_spec(cols):
  return pl.BlockSpec((ROWS, cols), lambda i: (i, 0))


def _full_spec(shape):
  nd = len(shape)
  return pl.BlockSpec(shape, lambda i: (0,) * nd)


def _agg_spec(group):
  return pl.BlockSpec((1, ROWS, GW), lambda i, g=group: (g, i, 0))


_DEGT_SPEC = pl.BlockSpec((ROWS, NC), lambda i: (i, 0))
_AGG_SPECS = [_agg_spec(g) for g in range(NG)]
_H_SPECS = [_row_spec(GW)] * NG
_H_SHAPES = [jax.ShapeDtypeStruct((N, GW), jnp.float32)] * NG

_t1 = pl.pallas_call(
    _t1_body,
    grid=(GRID,),
    in_specs=[_row_spec(D), _full_spec((D, D)), _DEGT_SPEC],
    out_specs=_H_SPECS,
    out_shape=_H_SHAPES,
)

_t2 = pl.pallas_call(
    _t2_body,
    grid=(GRID,),
    in_specs=_AGG_SPECS + _H_SPECS +
             [_DEGT_SPEC, _full_spec((D,)), _full_spec((D,)),
              _full_spec((D,)), _full_spec((D, D))],
    out_specs=_H_SPECS,
    out_shape=_H_SHAPES,
)

_t3 = pl.pallas_call(
    _t3_body,
    grid=(GRID,),
    in_specs=_AGG_SPECS + _H_SPECS +
             [_DEGT_SPEC, _full_spec((D,)), _full_spec((D,)),
              _full_spec((D,)), _row_spec(D)],
    out_specs=pl.BlockSpec((ROWS, D), lambda i: (i, 0)),
    out_shape=jax.ShapeDtypeStruct((N, D), jnp.float32),
)


@jax.jit
def kernel(x, edge_index, W1, b1, W2, b2, ln1_g, ln1_b, ln2_g, ln2_b):
  npad = E_PAD - E
  pad_src = jnp.arange(npad, dtype=jnp.int32) % N
  pad_dst = jnp.arange(npad, dtype=jnp.int32) % (NPAD - N) + N
  src = jnp.concatenate([edge_index[0], pad_src]).reshape(
      NS, AGG_NCHUNK, AGG_CHUNK)
  dst = jnp.concatenate([edge_index[1], pad_dst]).reshape(
      NS, AGG_NCHUNK, AGG_CHUNK)
  dst_deg = edge_index[1].reshape(NC, NS, DEG_NCHUNK, DEG_CHUNK)

  ones_c = jnp.ones((DEG_CHUNK,), jnp.float32)
  zeros_1 = jnp.zeros((STRIPE,), jnp.float32)
  zeros_2 = jnp.zeros((STRIPE, GW), jnp.float32)

  degp = _deg_kernel(dst_deg, ones_c, zeros_1)
  degt = degp.T

  h1 = _t1(x, W1, degt)
  agg1 = _agg_kernel(*h1, src, dst, zeros_2)
  h2 = _t2(agg1, agg1, *h1, degt, b1, ln1_g, ln1_b, W2)
  agg2 = _agg_kernel(*h2, src, dst, zeros_2)
  out = _t3(agg2, agg2, *h2, degt, b2, ln2_g, ln2_b, x)
  return out


# speedup vs baseline: 1.0185x; 1.0157x over previous
import functools

import jax
import jax.numpy as jnp
import numpy as np
from jax import lax
from jax.experimental import pallas as pl
from jax.experimental.pallas import tpu as pltpu
from jax.experimental.pallas import tpu_sc as plsc

N = 10000
E = 160000
D = 256
NG = 2
GW = D // NG
NPAD = 10240
STRIPE = NPAD // 16
NC = 2
NS = 16

AGG_CHUNK = 128
AGG_NCHUNK = 80
AGG_K = 8
AGG_NGRP = AGG_NCHUNK // AGG_K
EP_TILE = AGG_CHUNK * AGG_NCHUNK
E_PAD = EP_TILE * NS

DEG_CHUNK = 40
DEG_NCHUNK = (E // (NC * NS)) // DEG_CHUNK

ROWS = 2000
GRID = N // ROWS

_SQRT_HALF = np.float32(1.0 / np.sqrt(2.0))


def _sc_mesh():
  return plsc.VectorSubcoreMesh(
      core_axis_name="c", subcore_axis_name="s", num_cores=NC,
      num_subcores=NS)


@functools.partial(
    pl.kernel,
    out_type=jax.ShapeDtypeStruct((NC, NPAD), jnp.float32),
    mesh=_sc_mesh(),
    scratch_types=[
        pltpu.VMEM((DEG_NCHUNK, DEG_CHUNK), jnp.int32),
        pltpu.VMEM((DEG_CHUNK,), jnp.float32),
        pltpu.VMEM_SHARED((NPAD,), jnp.float32),
        pltpu.SemaphoreType.DMA,
    ],
)
def _deg_kernel(dst_hbm, ones_hbm, zeros_hbm, out_hbm, idx_v, ones_v,
                cnt_sh, sem):
  c = lax.axis_index("c")
  s = lax.axis_index("s")
  pltpu.sync_copy(dst_hbm.at[c, s], idx_v)
  pltpu.sync_copy(ones_hbm, ones_v)
  pltpu.sync_copy(zeros_hbm, cnt_sh.at[pl.ds(s * STRIPE, STRIPE)])
  plsc.subcore_barrier()

  def step(g, carry):
    base = g * 5
    for k in range(5):
      pltpu.async_copy(ones_v, cnt_sh.at[idx_v.at[base + k]], sem, add=True)
    for k in range(5):
      pltpu.make_async_copy(
          ones_v, cnt_sh.at[idx_v.at[base + k]], sem).wait()
    return carry

  lax.fori_loop(0, DEG_NCHUNK // 5, step, 0)
  plsc.subcore_barrier()
  pltpu.sync_copy(cnt_sh.at[pl.ds(s * STRIPE, STRIPE)],
                  out_hbm.at[c].at[pl.ds(s * STRIPE, STRIPE)])


@functools.partial(
    pl.kernel,
    out_type=jax.ShapeDtypeStruct((NG, NPAD, GW), jnp.float32),
    mesh=_sc_mesh(),
    scratch_types=[
        pltpu.VMEM((2, AGG_CHUNK), jnp.int32),
        pltpu.VMEM((AGG_NCHUNK, AGG_CHUNK), jnp.int32),
        pltpu.VMEM((2, AGG_CHUNK, GW), jnp.float32),
        pltpu.VMEM_SHARED((NPAD, GW), jnp.float32),
        pltpu.SemaphoreType.DMA,
        pltpu.SemaphoreType.DMA,
    ],
)
def _agg_kernel(hl_hbm, hr_hbm, src_hbm, dstidx_hbm,
                zeros_hbm, out_hbm, sidx_v, didx_v, buf_v, agg_sh,
                gsem, isem):
  c = lax.axis_index("c")
  s = lax.axis_index("s")
  pltpu.sync_copy(dstidx_hbm.at[s], didx_v)
  pltpu.sync_copy(zeros_hbm, agg_sh.at[pl.ds(s * STRIPE, STRIPE)])
  plsc.subcore_barrier()

  def pipeline(tbl):
    pltpu.sync_copy(src_hbm.at[s, 0], sidx_v.at[0])
    pltpu.async_copy(src_hbm.at[s, 1], sidx_v.at[1], isem)
    pltpu.async_copy(tbl.at[sidx_v.at[0]], buf_v.at[0], gsem)

    def step(j, carry):
      cur = lax.rem(j, 2)
      nxt = 1 - cur
      pltpu.make_async_copy(
          tbl.at[sidx_v.at[cur]], buf_v.at[cur], gsem).wait()

      @pl.when(j < AGG_NCHUNK - 1)
      def _():
        pltpu.make_async_copy(
            src_hbm.at[s, j + 1], sidx_v.at[nxt], isem).wait()
        pltpu.async_copy(tbl.at[sidx_v.at[nxt]], buf_v.at[nxt], gsem)

      @pl.when(j < AGG_NCHUNK - 2)
      def _():
        pltpu.async_copy(src_hbm.at[s, j + 2], sidx_v.at[cur], isem)

      pltpu.sync_copy(buf_v.at[cur], agg_sh.at[didx_v.at[j]], add=True)
      return carry

    lax.fori_loop(0, AGG_NCHUNK, step, 0)

  @pl.when(c == 0)
  def _():
    pipeline(hl_hbm)

  @pl.when(c == 1)
  def _():
    pipeline(hr_hbm)

  plsc.subcore_barrier()
  pltpu.sync_copy(agg_sh.at[pl.ds(s * STRIPE, STRIPE)],
                  out_hbm.at[c].at[pl.ds(s * STRIPE, STRIPE)])


def _dinv_from(degt):
  return lax.rsqrt(jnp.sum(degt, axis=-1, keepdims=True) + 1.0)


def _ln(v, g, b):
  mu = jnp.mean(v, axis=-1, keepdims=True)
  xc = v - mu
  var = jnp.mean(xc * xc, axis=-1, keepdims=True)
  return xc * lax.rsqrt(var + 1e-5) * g + b


def _gelu(v):
  return 0.5 * v * (1.0 + lax.erf(v * _SQRT_HALF))


def _t1_body(x_ref, w_ref, degt_ref, *h_refs):
  h = jnp.dot(x_ref[...], w_ref[...],
              preferred_element_type=jnp.float32,
              precision=lax.Precision.HIGHEST)
  h = h * _dinv_from(degt_ref[...])
  for g in range(NG):
    h_refs[g][...] = h[:, g * GW:(g + 1) * GW]


def _t2_body(a0, a1, h0, h1, degt_ref, b1_ref, g1_ref,
             bb1_ref, w2_ref, *o_refs):
  dinv = _dinv_from(degt_ref[...])
  agg = jnp.concatenate([a0[0], a1[0]], axis=-1)
  hp = jnp.concatenate([h0[...], h1[...]], axis=-1)
  conv = dinv * (agg + hp) + b1_ref[...]
  o1 = _gelu(_ln(conv, g1_ref[...], bb1_ref[...]))
  hh = jnp.dot(o1, w2_ref[...],
               preferred_element_type=jnp.float32,
               precision=lax.Precision.HIGHEST)
  hh = hh * dinv
  for g in range(NG):
    o_refs[g][...] = hh[:, g * GW:(g + 1) * GW]


def _t3_body(a0, a1, h0, h1, degt_ref, b2_ref, g2_ref,
             bb2_ref, x_ref, out_ref):
  dinv = _dinv_from(degt_ref[...])
  agg = jnp.concatenate([a0[0], a1[0]], axis=-1)
  hp = jnp.concatenate([h0[...], h1[...]], axis=-1)
  conv = dinv * (agg + hp) + b2_ref[...]
  out_ref[...] = _gelu(_ln(conv, g2_ref[...], bb2_ref[...]) + x_ref[...])


def _row_spec(cols):
  return pl.BlockSpec((ROWS, cols), lambda i: (i, 0))


def _full_spec(shape):
  nd = len(shape)
  return pl.BlockSpec(shape, lambda i: (0,) * nd)


def _agg_spec(group):
  return pl.BlockSpec((1, ROWS, GW), lambda i, g=group: (g, i, 0))


_DEGT_SPEC = pl.BlockSpec((ROWS, NC), lambda i: (i, 0))
_AGG_SPECS = [_agg_spec(g) for g in range(NG)]
_H_SPECS = [_row_spec(GW)] * NG
_H_SHAPES = [jax.ShapeDtypeStruct((N, GW), jnp.float32)] * NG

_t1 = pl.pallas_call(
    _t1_body,
    grid=(GRID,),
    in_specs=[_row_spec(D), _full_spec((D, D)), _DEGT_SPEC],
    out_specs=_H_SPECS,
    out_shape=_H_SHAPES,
)

_t2 = pl.pallas_call(
    _t2_body,
    grid=(GRID,),
    in_specs=_AGG_SPECS + _H_SPECS +
             [_DEGT_SPEC, _full_spec((D,)), _full_spec((D,)),
              _full_spec((D,)), _full_spec((D, D))],
    out_specs=_H_SPECS,
    out_shape=_H_SHAPES,
)

_t3 = pl.pallas_call(
    _t3_body,
    grid=(GRID,),
    in_specs=_AGG_SPECS + _H_SPECS +
             [_DEGT_SPEC, _full_spec((D,)), _full_spec((D,)),
              _full_spec((D,)), _row_spec(D)],
    out_specs=pl.BlockSpec((ROWS, D), lambda i: (i, 0)),
    out_shape=jax.ShapeDtypeStruct((N, D), jnp.float32),
)


@jax.jit
def kernel(x, edge_index, W1, b1, W2, b2, ln1_g, ln1_b, ln2_g, ln2_b):
  npad = E_PAD - E
  pad_src = jnp.arange(npad, dtype=jnp.int32) % N
  pad_dst = jnp.arange(npad, dtype=jnp.int32) % (NPAD - N) + N
  src = jnp.concatenate([edge_index[0], pad_src]).reshape(
      NS, AGG_NCHUNK, AGG_CHUNK)
  dst = jnp.concatenate([edge_index[1], pad_dst]).reshape(
      NS, AGG_NCHUNK, AGG_CHUNK)
  dst_deg = edge_index[1].reshape(NC, NS, DEG_NCHUNK, DEG_CHUNK)

  ones_c = jnp.ones((DEG_CHUNK,), jnp.float32)
  zeros_1 = jnp.zeros((STRIPE,), jnp.float32)
  zeros_2 = jnp.zeros((STRIPE, GW), jnp.float32)

  degp = _deg_kernel(dst_deg, ones_c, zeros_1)
  degt = degp.T

  h1 = _t1(x, W1, degt)
  agg1 = _agg_kernel(*h1, src, dst, zeros_2)
  h2 = _t2(agg1, agg1, *h1, degt, b1, ln1_g, ln1_b, W2)
  agg2 = _agg_kernel(*h2, src, dst, zeros_2)
  out = _t3(agg2, agg2, *h2, degt, b2, ln2_g, ln2_b, x)
  return out


# speedup vs baseline: 1.0275x; 1.0088x over previous
import functools

import jax
import jax.numpy as jnp
import numpy as np
from jax import lax
from jax.experimental import pallas as pl
from jax.experimental.pallas import tpu as pltpu
from jax.experimental.pallas import tpu_sc as plsc

N = 10000
E = 160000
D = 256
NG = 2
GW = D // NG
NPAD = 10240
STRIPE = NPAD // 16
NC = 2
NS = 16

AGG_CHUNK = 128
AGG_NCHUNK = 79
EP_TILE = AGG_CHUNK * AGG_NCHUNK
E_PAD = EP_TILE * NS

DEG_CHUNK = 40
DEG_NCHUNK = (E // (NC * NS)) // DEG_CHUNK

ROWS = 2000
GRID = N // ROWS

_SQRT_HALF = np.float32(1.0 / np.sqrt(2.0))


def _sc_mesh():
  return plsc.VectorSubcoreMesh(
      core_axis_name="c", subcore_axis_name="s", num_cores=NC,
      num_subcores=NS)


@functools.partial(
    pl.kernel,
    out_type=jax.ShapeDtypeStruct((NC, NPAD), jnp.float32),
    mesh=_sc_mesh(),
    scratch_types=[
        pltpu.VMEM((DEG_NCHUNK, DEG_CHUNK), jnp.int32),
        pltpu.VMEM((DEG_CHUNK,), jnp.float32),
        pltpu.VMEM_SHARED((NPAD,), jnp.float32),
        pltpu.SemaphoreType.DMA,
    ],
)
def _deg_kernel(dst_hbm, ones_hbm, zeros_hbm, out_hbm, idx_v, ones_v,
                cnt_sh, sem):
  c = lax.axis_index("c")
  s = lax.axis_index("s")
  pltpu.sync_copy(dst_hbm.at[c, s], idx_v)
  pltpu.sync_copy(ones_hbm, ones_v)
  pltpu.sync_copy(zeros_hbm, cnt_sh.at[pl.ds(s * STRIPE, STRIPE)])
  plsc.subcore_barrier()

  def step(g, carry):
    base = g * 5
    for k in range(5):
      pltpu.async_copy(ones_v, cnt_sh.at[idx_v.at[base + k]], sem, add=True)
    for k in range(5):
      pltpu.make_async_copy(
          ones_v, cnt_sh.at[idx_v.at[base + k]], sem).wait()
    return carry

  lax.fori_loop(0, DEG_NCHUNK // 5, step, 0)
  plsc.subcore_barrier()
  pltpu.sync_copy(cnt_sh.at[pl.ds(s * STRIPE, STRIPE)],
                  out_hbm.at[c].at[pl.ds(s * STRIPE, STRIPE)])


@functools.partial(
    pl.kernel,
    out_type=jax.ShapeDtypeStruct((NG, NPAD, GW), jnp.float32),
    mesh=_sc_mesh(),
    scratch_types=[
        pltpu.VMEM((2, AGG_CHUNK), jnp.int32),
        pltpu.VMEM((AGG_NCHUNK, AGG_CHUNK), jnp.int32),
        pltpu.VMEM((2, AGG_CHUNK, GW), jnp.float32),
        pltpu.VMEM_SHARED((NPAD, GW), jnp.float32),
        pltpu.SemaphoreType.DMA,
        pltpu.SemaphoreType.DMA,
    ],
)
def _agg_kernel(hl_hbm, hr_hbm, src_hbm, dstidx_hbm,
                zeros_hbm, out_hbm, sidx_v, didx_v, buf_v, agg_sh,
                gsem, isem):
  c = lax.axis_index("c")
  s = lax.axis_index("s")
  pltpu.sync_copy(dstidx_hbm.at[s], didx_v)
  pltpu.sync_copy(zeros_hbm, agg_sh.at[pl.ds(s * STRIPE, STRIPE)])
  plsc.subcore_barrier()

  def pipeline(tbl):
    pltpu.sync_copy(src_hbm.at[s, 0], sidx_v.at[0])
    pltpu.async_copy(src_hbm.at[s, 1], sidx_v.at[1], isem)
    pltpu.async_copy(tbl.at[sidx_v.at[0]], buf_v.at[0], gsem)

    def step(j, carry):
      cur = lax.rem(j, 2)
      nxt = 1 - cur
      pltpu.make_async_copy(
          tbl.at[sidx_v.at[cur]], buf_v.at[cur], gsem).wait()

      @pl.when(j < AGG_NCHUNK - 1)
      def _():
        pltpu.make_async_copy(
            src_hbm.at[s, j + 1], sidx_v.at[nxt], isem).wait()
        pltpu.async_copy(tbl.at[sidx_v.at[nxt]], buf_v.at[nxt], gsem)

      @pl.when(j < AGG_NCHUNK - 2)
      def _():
        pltpu.async_copy(src_hbm.at[s, j + 2], sidx_v.at[cur], isem)

      pltpu.sync_copy(buf_v.at[cur], agg_sh.at[didx_v.at[j]], add=True)
      return carry

    lax.fori_loop(0, AGG_NCHUNK, step, 0)

  @pl.when(c == 0)
  def _():
    pipeline(hl_hbm)

  @pl.when(c == 1)
  def _():
    pipeline(hr_hbm)

  plsc.subcore_barrier()
  pltpu.sync_copy(agg_sh.at[pl.ds(s * STRIPE, STRIPE)],
                  out_hbm.at[c].at[pl.ds(s * STRIPE, STRIPE)])


def _dinv_from(degt):
  return lax.rsqrt(jnp.sum(degt, axis=-1, keepdims=True) + 1.0)


def _ln(v, g, b):
  mu = jnp.mean(v, axis=-1, keepdims=True)
  xc = v - mu
  var = jnp.mean(xc * xc, axis=-1, keepdims=True)
  return xc * lax.rsqrt(var + 1e-5) * g + b


def _gelu(v):
  return 0.5 * v * (1.0 + lax.erf(v * _SQRT_HALF))


def _t1_body(x_ref, w_ref, degt_ref, *h_refs):
  h = jnp.dot(x_ref[...], w_ref[...],
              preferred_element_type=jnp.float32,
              precision=lax.Precision.HIGHEST)
  h = h * _dinv_from(degt_ref[...])
  for g in range(NG):
    h_refs[g][...] = h[:, g * GW:(g + 1) * GW]


def _t2_body(a0, a1, h0, h1, degt_ref, b1_ref, g1_ref,
             bb1_ref, w2_ref, *o_refs):
  dinv = _dinv_from(degt_ref[...])
  agg = jnp.concatenate([a0[0], a1[0]], axis=-1)
  hp = jnp.concatenate([h0[...], h1[...]], axis=-1)
  conv = dinv * (agg + hp) + b1_ref[...]
  o1 = _gelu(_ln(conv, g1_ref[...], bb1_ref[...]))
  hh = jnp.dot(o1, w2_ref[...],
               preferred_element_type=jnp.float32,
               precision=lax.Precision.HIGHEST)
  hh = hh * dinv
  for g in range(NG):
    o_refs[g][...] = hh[:, g * GW:(g + 1) * GW]


def _t3_body(a0, a1, h0, h1, degt_ref, b2_ref, g2_ref,
             bb2_ref, x_ref, out_ref):
  dinv = _dinv_from(degt_ref[...])
  agg = jnp.concatenate([a0[0], a1[0]], axis=-1)
  hp = jnp.concatenate([h0[...], h1[...]], axis=-1)
  conv = dinv * (agg + hp) + b2_ref[...]
  out_ref[...] = _gelu(_ln(conv, g2_ref[...], bb2_ref[...]) + x_ref[...])


def _row_spec(cols):
  return pl.BlockSpec((ROWS, cols), lambda i: (i, 0))


def _full_spec(shape):
  nd = len(shape)
  return pl.BlockSpec(shape, lambda i: (0,) * nd)


def _agg_spec(group):
  return pl.BlockSpec((1, ROWS, GW), lambda i, g=group: (g, i, 0))


_DEGT_SPEC = pl.BlockSpec((ROWS, NC), lambda i: (i, 0))
_AGG_SPECS = [_agg_spec(g) for g in range(NG)]
_H_SPECS = [_row_spec(GW)] * NG
_H_SHAPES = [jax.ShapeDtypeStruct((N, GW), jnp.float32)] * NG

_t1 = pl.pallas_call(
    _t1_body,
    grid=(GRID,),
    in_specs=[_row_spec(D), _full_spec((D, D)), _DEGT_SPEC],
    out_specs=_H_SPECS,
    out_shape=_H_SHAPES,
)

_t2 = pl.pallas_call(
    _t2_body,
    grid=(GRID,),
    in_specs=_AGG_SPECS + _H_SPECS +
             [_DEGT_SPEC, _full_spec((D,)), _full_spec((D,)),
              _full_spec((D,)), _full_spec((D, D))],
    out_specs=_H_SPECS,
    out_shape=_H_SHAPES,
)

_t3 = pl.pallas_call(
    _t3_body,
    grid=(GRID,),
    in_specs=_AGG_SPECS + _H_SPECS +
             [_DEGT_SPEC, _full_spec((D,)), _full_spec((D,)),
              _full_spec((D,)), _row_spec(D)],
    out_specs=pl.BlockSpec((ROWS, D), lambda i: (i, 0)),
    out_shape=jax.ShapeDtypeStruct((N, D), jnp.float32),
)


@jax.jit
def kernel(x, edge_index, W1, b1, W2, b2, ln1_g, ln1_b, ln2_g, ln2_b):
  npad = E_PAD - E
  pad_src = jnp.arange(npad, dtype=jnp.int32) % N
  pad_dst = jnp.arange(npad, dtype=jnp.int32) % (NPAD - N) + N
  src = jnp.concatenate([edge_index[0], pad_src]).reshape(
      NS, AGG_NCHUNK, AGG_CHUNK)
  dst = jnp.concatenate([edge_index[1], pad_dst]).reshape(
      NS, AGG_NCHUNK, AGG_CHUNK)
  dst_deg = edge_index[1].reshape(NC, NS, DEG_NCHUNK, DEG_CHUNK)

  ones_c = jnp.ones((DEG_CHUNK,), jnp.float32)
  zeros_1 = jnp.zeros((STRIPE,), jnp.float32)
  zeros_2 = jnp.zeros((STRIPE, GW), jnp.float32)

  degp = _deg_kernel(dst_deg, ones_c, zeros_1)
  degt = degp.T

  h1 = _t1(x, W1, degt)
  agg1 = _agg_kernel(*h1, src, dst, zeros_2)
  h2 = _t2(agg1, agg1, *h1, degt, b1, ln1_g, ln1_b, W2)
  agg2 = _agg_kernel(*h2, src, dst, zeros_2)
  out = _t3(agg2, agg2, *h2, degt, b2, ln2_g, ln2_b, x)
  return out
